# baseline (speedup 1.0000x reference)
import functools

import jax
import jax.numpy as jnp
from jax import lax
from jax.experimental import pallas as pl
from jax.experimental.pallas import tpu as pltpu
from jax.experimental.pallas import tpu_sc as plsc

BATCH = 16384
D = 64
NC = 2
NS = 16
NW = NC * NS
B_PER_W = BATCH // NW
CHUNK = 128
NCHUNK = B_PER_W // CHUNK


def _bprmf_body(user_hbm, pos_hbm, neg_hbm, utab_hbm, itab_hbm,
                out_u, out_p, out_n,
                idx_u, idx_p, idx_n, rows_u, rows_p, rows_n, sem):
    cid = lax.axis_index("c")
    sid = lax.axis_index("s")
    wid = sid * NC + cid
    base = wid * B_PER_W

    pltpu.sync_copy(user_hbm.at[wid], idx_u)
    pltpu.sync_copy(pos_hbm.at[wid], idx_p)
    pltpu.sync_copy(neg_hbm.at[wid], idx_n)

    copies = []
    for c in range(NCHUNK):
        dst = pl.ds(c * CHUNK, CHUNK)
        copies.append(pltpu.async_copy(utab_hbm.at[idx_u.at[c]],
                                       rows_u.at[dst], sem))
        copies.append(pltpu.async_copy(itab_hbm.at[idx_p.at[c]],
                                       rows_p.at[dst], sem))
        copies.append(pltpu.async_copy(itab_hbm.at[idx_n.at[c]],
                                       rows_n.at[dst], sem))
    for cp in copies:
        cp.wait()

    dst = pl.ds(base, B_PER_W)
    pltpu.sync_copy(rows_u, out_u.at[dst])
    pltpu.sync_copy(rows_p, out_p.at[dst])
    pltpu.sync_copy(rows_n, out_n.at[dst])


@jax.jit
def _bprmf_call(user, pos_item, neg_item, user_table, item_table):
    mesh = plsc.VectorSubcoreMesh(core_axis_name="c", subcore_axis_name="s")
    out = jax.ShapeDtypeStruct((BATCH, D), jnp.float32)
    fn = functools.partial(
        pl.kernel,
        mesh=mesh,
        out_type=(out, out, out),
        scratch_types=[
            pltpu.VMEM((NCHUNK, CHUNK), jnp.int32),
            pltpu.VMEM((NCHUNK, CHUNK), jnp.int32),
            pltpu.VMEM((NCHUNK, CHUNK), jnp.int32),
            pltpu.VMEM((B_PER_W, D), jnp.float32),
            pltpu.VMEM((B_PER_W, D), jnp.float32),
            pltpu.VMEM((B_PER_W, D), jnp.float32),
            pltpu.SemaphoreType.DMA,
        ],
        compiler_params=pltpu.CompilerParams(use_tc_tiling_on_sc=False),
    )(_bprmf_body)
    return fn(user.reshape(NW, NCHUNK, CHUNK),
              pos_item.reshape(NW, NCHUNK, CHUNK),
              neg_item.reshape(NW, NCHUNK, CHUNK),
              user_table, item_table)


def kernel(user, pos_item, neg_item, user_table, item_table):
    return _bprmf_call(user, pos_item, neg_item, user_table, item_table)
